# baseline (speedup 1.0000x reference)
import jax
import jax.numpy as jnp
from jax import lax
from jax.experimental import pallas as pl
from jax.experimental.pallas import tpu as pltpu
from jax.experimental.pallas import tpu_sc as plsc

N_NODES = 10000
N_EDGES = 320000
D_FEAT = 128
W_PAD = 144
NC, NS = 2, 16
NW = NC * NS
E_PER_TILE = N_EDGES // NW
CHUNK = 100
N_SEC = 5
SEC_CHUNKS = 20
ROWS_PER_TILE = N_NODES // NS


def _sc_body(x_hbm, ei_hbm, zeros_hbm, out0_hbm, out1_hbm,
             isec0, isec1, rows0_v, rows1_v, acc_sh,
             isem0, isem1, gsem0, gsem1):
    c = lax.axis_index("c")
    s = lax.axis_index("s")
    wid = c * NS + s

    isecs, isems = (isec0, isec1), (isem0, isem1)
    bufs, gsems = (rows0_v, rows1_v), (gsem0, gsem1)

    pltpu.async_copy(ei_hbm.at[wid, 0], isec0, isem0)
    row0 = pl.multiple_of(s * ROWS_PER_TILE, 8)
    pltpu.sync_copy(zeros_hbm, acc_sh.at[pl.ds(row0, ROWS_PER_TILE)])
    plsc.subcore_barrier()

    for sec in range(N_SEC):
        ib = isecs[sec % 2]
        if sec + 1 < N_SEC:
            pltpu.async_copy(ei_hbm.at[wid, sec + 1],
                             isecs[(sec + 1) % 2], isems[(sec + 1) % 2])
        pltpu.make_async_copy(ei_hbm.at[wid, sec], ib, isems[sec % 2]).wait()

        pltpu.async_copy(x_hbm.at[ib.at[0, 0]], rows0_v, gsem0)
        pltpu.async_copy(x_hbm.at[ib.at[1, 0]], rows1_v, gsem1)

        def body(j, carry):
            for b in (0, 1):
                k = 2 * j + b
                pltpu.make_async_copy(x_hbm.at[ib.at[k, 0]],
                                      bufs[b], gsems[b]).wait()
                pltpu.sync_copy(bufs[b], acc_sh.at[ib.at[k, 1]], add=True)
                nxt = k + 2

                @pl.when(nxt < SEC_CHUNKS)
                def _():
                    pltpu.async_copy(x_hbm.at[ib.at[nxt, 0]], bufs[b], gsems[b])
            return carry

        lax.fori_loop(0, SEC_CHUNKS // 2, body, 0)

    plsc.subcore_barrier()

    @pl.when(c == 0)
    def _():
        pltpu.sync_copy(acc_sh.at[pl.ds(row0, ROWS_PER_TILE)],
                        out0_hbm.at[pl.ds(row0, ROWS_PER_TILE)])

    @pl.when(c == 1)
    def _():
        pltpu.sync_copy(acc_sh.at[pl.ds(row0, ROWS_PER_TILE)],
                        out1_hbm.at[pl.ds(row0, ROWS_PER_TILE)])


_sc_call = pl.kernel(
    _sc_body,
    out_type=(
        jax.ShapeDtypeStruct((N_NODES, W_PAD), jnp.float32),
        jax.ShapeDtypeStruct((N_NODES, W_PAD), jnp.float32),
    ),
    mesh=plsc.VectorSubcoreMesh(core_axis_name="c", subcore_axis_name="s"),
    compiler_params=pltpu.CompilerParams(use_tc_tiling_on_sc=False),
    scratch_types=(
        pltpu.VMEM((SEC_CHUNKS, 2, CHUNK), jnp.int32),
        pltpu.VMEM((SEC_CHUNKS, 2, CHUNK), jnp.int32),
        pltpu.VMEM((CHUNK, W_PAD), jnp.float32),
        pltpu.VMEM((CHUNK, W_PAD), jnp.float32),
        pltpu.VMEM_SHARED((N_NODES, W_PAD), jnp.float32),
        pltpu.SemaphoreType.DMA,
        pltpu.SemaphoreType.DMA,
        pltpu.SemaphoreType.DMA,
        pltpu.SemaphoreType.DMA,
    ),
)


def _combine_body(a_ref, b_ref, o_ref):
    s = a_ref[:, :D_FEAT] + b_ref[:, :D_FEAT]
    d = a_ref[:, D_FEAT:D_FEAT + 1] + b_ref[:, D_FEAT:D_FEAT + 1]
    o_ref[:, :] = s / jnp.maximum(d, 1e-8)


_combine = pl.pallas_call(
    _combine_body,
    out_shape=jax.ShapeDtypeStruct((N_NODES, D_FEAT), jnp.float32),
)


@jax.jit
def kernel(x, edge_index):
    pad = jnp.concatenate(
        [jnp.ones((N_NODES, 1), jnp.float32),
         jnp.zeros((N_NODES, W_PAD - D_FEAT - 1), jnp.float32)], axis=1)
    x_pad = jnp.concatenate([x, pad], axis=1)
    zeros = jnp.zeros((ROWS_PER_TILE, W_PAD), jnp.float32)
    ei = edge_index.astype(jnp.int32).reshape(2, NW, N_SEC, SEC_CHUNKS, CHUNK)
    ei = jnp.transpose(ei, (1, 2, 3, 0, 4))
    p0, p1 = _sc_call(x_pad, ei, zeros)
    return _combine(p0, p1)


# speedup vs baseline: 1.2067x; 1.2067x over previous
import jax
import jax.numpy as jnp
from jax import lax
from jax.experimental import pallas as pl
from jax.experimental.pallas import tpu as pltpu
from jax.experimental.pallas import tpu_sc as plsc

N_NODES = 10000
N_EDGES = 320000
D_FEAT = 128
DEG_W = 16
NC, NS = 2, 16
NW = NC * NS
E_PER_TILE = N_EDGES // NW
CHUNK = 100
N_SEC = 5
SEC_CHUNKS = 20
ROWS_PER_TILE = N_NODES // NS


def _sc_body(x_hbm, ei_hbm, zeros_hbm, zerod_hbm, ones_hbm,
             out0_hbm, out1_hbm, deg0_hbm, deg1_hbm,
             src0_v, src1_v, dst0_v, dst1_v, rows0_v, rows1_v, ones_v,
             acc_sh, deg_sh, isem0, isem1, gsem0, gsem1):
    c = lax.axis_index("c")
    s = lax.axis_index("s")
    wid = c * NS + s

    srcs, dsts, isems = (src0_v, src1_v), (dst0_v, dst1_v), (isem0, isem1)
    bufs, gsems = (rows0_v, rows1_v), (gsem0, gsem1)

    pltpu.async_copy(ei_hbm.at[0, wid, 0], src0_v, isem0)
    pltpu.async_copy(ei_hbm.at[1, wid, 0], dst0_v, isem0)
    pltpu.sync_copy(ones_hbm, ones_v)
    row0 = pl.multiple_of(s * ROWS_PER_TILE, 8)
    pltpu.sync_copy(zeros_hbm, acc_sh.at[pl.ds(row0, ROWS_PER_TILE)])
    pltpu.sync_copy(zerod_hbm, deg_sh.at[pl.ds(row0, ROWS_PER_TILE)])
    plsc.subcore_barrier()

    for sec in range(N_SEC):
        sb, db = srcs[sec % 2], dsts[sec % 2]
        if sec + 1 < N_SEC:
            nb = (sec + 1) % 2
            pltpu.async_copy(ei_hbm.at[0, wid, sec + 1], srcs[nb], isems[nb])
            pltpu.async_copy(ei_hbm.at[1, wid, sec + 1], dsts[nb], isems[nb])
        pltpu.make_async_copy(ei_hbm.at[0, wid, sec], sb, isems[sec % 2]).wait()
        pltpu.make_async_copy(ei_hbm.at[1, wid, sec], db, isems[sec % 2]).wait()

        pltpu.async_copy(x_hbm.at[sb.at[0]], rows0_v, gsem0)
        pltpu.async_copy(x_hbm.at[sb.at[1]], rows1_v, gsem1)

        def body(j, carry):
            for b in (0, 1):
                k = 2 * j + b
                pltpu.make_async_copy(x_hbm.at[sb.at[k]],
                                      bufs[b], gsems[b]).wait()
                pltpu.sync_copy(bufs[b], acc_sh.at[db.at[k]], add=True)
                pltpu.sync_copy(ones_v, deg_sh.at[db.at[k]], add=True)
                nxt = k + 2

                @pl.when(nxt < SEC_CHUNKS)
                def _():
                    pltpu.async_copy(x_hbm.at[sb.at[nxt]], bufs[b], gsems[b])
            return carry

        lax.fori_loop(0, SEC_CHUNKS // 2, body, 0)

    plsc.subcore_barrier()

    @pl.when(c == 0)
    def _():
        pltpu.sync_copy(acc_sh.at[pl.ds(row0, ROWS_PER_TILE)],
                        out0_hbm.at[pl.ds(row0, ROWS_PER_TILE)])
        pltpu.sync_copy(deg_sh.at[pl.ds(row0, ROWS_PER_TILE)],
                        deg0_hbm.at[pl.ds(row0, ROWS_PER_TILE)])

    @pl.when(c == 1)
    def _():
        pltpu.sync_copy(acc_sh.at[pl.ds(row0, ROWS_PER_TILE)],
                        out1_hbm.at[pl.ds(row0, ROWS_PER_TILE)])
        pltpu.sync_copy(deg_sh.at[pl.ds(row0, ROWS_PER_TILE)],
                        deg1_hbm.at[pl.ds(row0, ROWS_PER_TILE)])


_sc_call = pl.kernel(
    _sc_body,
    out_type=(
        jax.ShapeDtypeStruct((N_NODES, D_FEAT), jnp.float32),
        jax.ShapeDtypeStruct((N_NODES, D_FEAT), jnp.float32),
        jax.ShapeDtypeStruct((N_NODES, DEG_W), jnp.float32),
        jax.ShapeDtypeStruct((N_NODES, DEG_W), jnp.float32),
    ),
    mesh=plsc.VectorSubcoreMesh(core_axis_name="c", subcore_axis_name="s"),
    compiler_params=pltpu.CompilerParams(use_tc_tiling_on_sc=False),
    scratch_types=(
        pltpu.VMEM((SEC_CHUNKS, CHUNK), jnp.int32),
        pltpu.VMEM((SEC_CHUNKS, CHUNK), jnp.int32),
        pltpu.VMEM((SEC_CHUNKS, CHUNK), jnp.int32),
        pltpu.VMEM((SEC_CHUNKS, CHUNK), jnp.int32),
        pltpu.VMEM((CHUNK, D_FEAT), jnp.float32),
        pltpu.VMEM((CHUNK, D_FEAT), jnp.float32),
        pltpu.VMEM((CHUNK, DEG_W), jnp.float32),
        pltpu.VMEM_SHARED((N_NODES, D_FEAT), jnp.float32),
        pltpu.VMEM_SHARED((N_NODES, DEG_W), jnp.float32),
        pltpu.SemaphoreType.DMA,
        pltpu.SemaphoreType.DMA,
        pltpu.SemaphoreType.DMA,
        pltpu.SemaphoreType.DMA,
    ),
)


def _combine_body(a_ref, b_ref, da_ref, db_ref, o_ref):
    s = a_ref[...] + b_ref[...]
    d = da_ref[:, :1] + db_ref[:, :1]
    o_ref[...] = s / jnp.maximum(d, 1e-8)


_BLK = 2000

_combine = pl.pallas_call(
    _combine_body,
    grid=(N_NODES // _BLK,),
    in_specs=[
        pl.BlockSpec((_BLK, D_FEAT), lambda i: (i, 0)),
        pl.BlockSpec((_BLK, D_FEAT), lambda i: (i, 0)),
        pl.BlockSpec((_BLK, DEG_W), lambda i: (i, 0)),
        pl.BlockSpec((_BLK, DEG_W), lambda i: (i, 0)),
    ],
    out_specs=pl.BlockSpec((_BLK, D_FEAT), lambda i: (i, 0)),
    out_shape=jax.ShapeDtypeStruct((N_NODES, D_FEAT), jnp.float32),
)


@jax.jit
def kernel(x, edge_index):
    zeros = jnp.zeros((ROWS_PER_TILE, D_FEAT), jnp.float32)
    zerod = jnp.zeros((ROWS_PER_TILE, DEG_W), jnp.float32)
    ones = jnp.ones((CHUNK, DEG_W), jnp.float32)
    ei = edge_index.astype(jnp.int32).reshape(2, NW, N_SEC, SEC_CHUNKS, CHUNK)
    p0, p1, d0, d1 = _sc_call(x, ei, zeros, zerod, ones)
    return _combine(p0, p1, d0, d1)


# speedup vs baseline: 1.2949x; 1.0731x over previous
import jax
import jax.numpy as jnp
from jax import lax
from jax.experimental import pallas as pl
from jax.experimental.pallas import tpu as pltpu
from jax.experimental.pallas import tpu_sc as plsc

N_NODES = 10000
N_EDGES = 320000
D_FEAT = 128
DEG_W = 16
NC, NS = 2, 16
NW = NC * NS
CHUNK = 128
N_CHUNKS = N_EDGES // CHUNK
CH_PER_TILE = N_CHUNKS // NW
N_TAIL = N_CHUNKS - CH_PER_TILE * NW
SEC_CHUNKS = 6
N_SEC = CH_PER_TILE // SEC_CHUNKS
ROWS_PER_TILE = N_NODES // NS


def _sc_body(x_hbm, ei_hbm, zeros_hbm, zerod_hbm, ones_hbm,
             out0_hbm, out1_hbm, deg0_hbm, deg1_hbm,
             isec0, isec1, rows0_v, rows1_v, ones_v,
             acc_sh, deg_sh, isem0, isem1, gsem0, gsem1):
    c = lax.axis_index("c")
    s = lax.axis_index("s")
    wid = c * NS + s
    base = wid * CH_PER_TILE

    isecs, isems = (isec0, isec1), (isem0, isem1)
    bufs, gsems = (rows0_v, rows1_v), (gsem0, gsem1)

    pltpu.async_copy(ei_hbm.at[pl.ds(base, SEC_CHUNKS)], isec0, isem0)
    pltpu.sync_copy(ones_hbm, ones_v)
    row0 = pl.multiple_of(s * ROWS_PER_TILE, 8)
    pltpu.sync_copy(zeros_hbm, acc_sh.at[pl.ds(row0, ROWS_PER_TILE)])
    pltpu.sync_copy(zerod_hbm, deg_sh.at[pl.ds(row0, ROWS_PER_TILE)])
    plsc.subcore_barrier()

    for sec in range(N_SEC):
        ib = isecs[sec % 2]
        if sec + 1 < N_SEC:
            nb = (sec + 1) % 2
            pltpu.async_copy(
                ei_hbm.at[pl.ds(base + (sec + 1) * SEC_CHUNKS, SEC_CHUNKS)],
                isecs[nb], isems[nb])
        pltpu.make_async_copy(ei_hbm.at[pl.ds(base, SEC_CHUNKS)],
                              ib, isems[sec % 2]).wait()

        pltpu.async_copy(x_hbm.at[ib.at[0, 0]], rows0_v, gsem0)
        pltpu.async_copy(x_hbm.at[ib.at[1, 0]], rows1_v, gsem1)

        def body(j, carry):
            for b in (0, 1):
                k = 2 * j + b
                pltpu.make_async_copy(x_hbm.at[ib.at[k, 0]],
                                      bufs[b], gsems[b]).wait()
                pltpu.sync_copy(bufs[b], acc_sh.at[ib.at[k, 1]], add=True)
                pltpu.sync_copy(ones_v, deg_sh.at[ib.at[k, 1]], add=True)
                nxt = k + 2

                @pl.when(nxt < SEC_CHUNKS)
                def _():
                    pltpu.async_copy(x_hbm.at[ib.at[nxt, 0]], bufs[b], gsems[b])
            return carry

        lax.fori_loop(0, SEC_CHUNKS // 2, body, 0)

    @pl.when(wid < N_TAIL)
    def _():
        cx = NW * CH_PER_TILE + wid
        pltpu.sync_copy(ei_hbm.at[cx], isec0.at[0])
        pltpu.async_copy(x_hbm.at[isec0.at[0, 0]], rows0_v, gsem0).wait()
        pltpu.sync_copy(rows0_v, acc_sh.at[isec0.at[0, 1]], add=True)
        pltpu.sync_copy(ones_v, deg_sh.at[isec0.at[0, 1]], add=True)

    plsc.subcore_barrier()

    @pl.when(c == 0)
    def _():
        pltpu.sync_copy(acc_sh.at[pl.ds(row0, ROWS_PER_TILE)],
                        out0_hbm.at[pl.ds(row0, ROWS_PER_TILE)])
        pltpu.sync_copy(deg_sh.at[pl.ds(row0, ROWS_PER_TILE)],
                        deg0_hbm.at[pl.ds(row0, ROWS_PER_TILE), pl.ds(0, DEG_W)])

    @pl.when(c == 1)
    def _():
        pltpu.sync_copy(acc_sh.at[pl.ds(row0, ROWS_PER_TILE)],
                        out1_hbm.at[pl.ds(row0, ROWS_PER_TILE)])
        pltpu.sync_copy(deg_sh.at[pl.ds(row0, ROWS_PER_TILE)],
                        deg1_hbm.at[pl.ds(row0, ROWS_PER_TILE), pl.ds(0, DEG_W)])


_sc_call = pl.kernel(
    _sc_body,
    out_type=(
        jax.ShapeDtypeStruct((N_NODES, D_FEAT), jnp.float32),
        jax.ShapeDtypeStruct((N_NODES, D_FEAT), jnp.float32),
        jax.ShapeDtypeStruct((N_NODES, D_FEAT), jnp.float32),
        jax.ShapeDtypeStruct((N_NODES, D_FEAT), jnp.float32),
    ),
    mesh=plsc.VectorSubcoreMesh(core_axis_name="c", subcore_axis_name="s"),
    compiler_params=pltpu.CompilerParams(use_tc_tiling_on_sc=False),
    scratch_types=(
        pltpu.VMEM((SEC_CHUNKS, 2, CHUNK), jnp.int32),
        pltpu.VMEM((SEC_CHUNKS, 2, CHUNK), jnp.int32),
        pltpu.VMEM((CHUNK, D_FEAT), jnp.float32),
        pltpu.VMEM((CHUNK, D_FEAT), jnp.float32),
        pltpu.VMEM((CHUNK, DEG_W), jnp.float32),
        pltpu.VMEM_SHARED((N_NODES, D_FEAT), jnp.float32),
        pltpu.VMEM_SHARED((N_NODES, DEG_W), jnp.float32),
        pltpu.SemaphoreType.DMA,
        pltpu.SemaphoreType.DMA,
        pltpu.SemaphoreType.DMA,
        pltpu.SemaphoreType.DMA,
    ),
)


_BLK = 2000


def _combine_body(a_ref, b_ref, da_ref, db_ref, o_ref):
    s = a_ref[...] + b_ref[...]
    d = da_ref[:, :1] + db_ref[:, :1]
    o_ref[...] = s / jnp.maximum(d, 1e-8)


_combine = pl.pallas_call(
    _combine_body,
    grid=(N_NODES // _BLK,),
    in_specs=[
        pl.BlockSpec((_BLK, D_FEAT), lambda i: (i, 0)),
        pl.BlockSpec((_BLK, D_FEAT), lambda i: (i, 0)),
        pl.BlockSpec((_BLK, D_FEAT), lambda i: (i, 0)),
        pl.BlockSpec((_BLK, D_FEAT), lambda i: (i, 0)),
    ],
    out_specs=pl.BlockSpec((_BLK, D_FEAT), lambda i: (i, 0)),
    out_shape=jax.ShapeDtypeStruct((N_NODES, D_FEAT), jnp.float32),
)


@jax.jit
def kernel(x, edge_index):
    zeros = jnp.zeros((ROWS_PER_TILE, D_FEAT), jnp.float32)
    zerod = jnp.zeros((ROWS_PER_TILE, DEG_W), jnp.float32)
    ones = jnp.ones((CHUNK, DEG_W), jnp.float32)
    ei = jnp.transpose(
        edge_index.astype(jnp.int32).reshape(2, N_CHUNKS, CHUNK), (1, 0, 2))
    p0, p1, d0, d1 = _sc_call(x, ei, zeros, zerod, ones)
    return _combine(p0, p1, d0, d1)
